# Initial kernel scaffold; baseline (speedup 1.0000x reference)
import jax
import jax.numpy as jnp
from jax.experimental import pallas as pl


def kernel(x, edge_index, W, b):
    raise NotImplementedError("write your pallas kernel here")


# speedup vs baseline: 20.1950x; 20.1950x over previous
import functools

import jax
import jax.numpy as jnp
from jax import lax
from jax.experimental import pallas as pl
from jax.experimental.pallas import tpu as pltpu
from jax.experimental.pallas import tpu_sc as plsc

N_NODES = 10000
N_PAD = 10240
D_IN = 128
C_OUT = 64
C_HALF = 32
E_EDGES = 320000
K_HOPS = 16
ALPHA = 0.05
COEF = (1.0 - ALPHA) / K_HOPS

EC = 128
EROWS = 2560
EROWS_PER_TILE = EROWS // 16
EROWS_PER_TILE32 = EROWS // 32
NPT = N_PAD // 16
NCH = NPT // 128

_mesh = plsc.VectorSubcoreMesh(
    core_axis_name="c", subcore_axis_name="s", num_cores=2, num_subcores=16)


def _fill(ref, n16, val, dtype):
  def body(i, _):
    ref[pl.ds(i * 16, 16)] = jnp.full((16,), val, dtype)
    return 0
  lax.fori_loop(0, n16, body, 0)


def _fill2d(ref, rows, cols, val, dtype):
  def body(i, _):
    r = i // (cols // 16)
    g = i % (cols // 16)
    ref[r, pl.ds(g * 16, 16)] = jnp.full((16,), val, dtype)
    return 0
  lax.fori_loop(0, rows * (cols // 16), body, 0)


@functools.partial(
    pl.kernel,
    out_type=jax.ShapeDtypeStruct((2, N_PAD), jnp.float32),
    mesh=_mesh,
    scratch_types=[
        pltpu.VMEM((EROWS_PER_TILE32, EC), jnp.int32),
        pltpu.VMEM((EC,), jnp.float32),
        pltpu.VMEM((NPT,), jnp.float32),
        pltpu.VMEM_SHARED((N_PAD,), jnp.float32),
    ],
)
def _sc_degree(col_hbm, deg_out, colb, ones, zb, degsh):
  cid = lax.axis_index("c")
  sid = lax.axis_index("s")
  tid = sid * 2 + cid
  pltpu.sync_copy(col_hbm.at[pl.ds(tid * EROWS_PER_TILE32, EROWS_PER_TILE32)],
                  colb)
  _fill(ones, EC // 16, 1.0, jnp.float32)
  _fill(zb, NPT // 16, 0.0, jnp.float32)
  pltpu.sync_copy(zb, degsh.at[pl.ds(sid * NPT, NPT)])
  plsc.subcore_barrier()

  def chunk(j, _):
    pltpu.sync_copy(ones, degsh.at[colb.at[j]], add=True)
    return 0
  lax.fori_loop(0, EROWS_PER_TILE32, chunk, 0)
  plsc.subcore_barrier()
  pltpu.sync_copy(degsh.at[pl.ds(sid * NPT, NPT)],
                  deg_out.at[cid, pl.ds(sid * NPT, NPT)])


def _tc_prep_body(x_ref, w_ref, dp_ref, y0_ref, u0_ref, d1_ref, dsq_ref):
  y0 = jnp.dot(x_ref[...], w_ref[...], preferred_element_type=jnp.float32)
  deg = dp_ref[:, 0:1] + dp_ref[:, 1:2] + 1.0
  dinv = lax.rsqrt(deg)
  y0_ref[...] = y0
  u0_ref[...] = y0 * dinv
  d1_ref[...] = dinv * dinv
  dsq_ref[...] = deg * dinv


def _tc_prep(x_pad, W, deg_t):
  blk = 512
  grid = (N_PAD // blk,)
  return pl.pallas_call(
      _tc_prep_body,
      grid=grid,
      in_specs=[
          pl.BlockSpec((blk, D_IN), lambda i: (i, 0)),
          pl.BlockSpec((D_IN, C_OUT), lambda i: (0, 0)),
          pl.BlockSpec((blk, 2), lambda i: (i, 0)),
      ],
      out_specs=[
          pl.BlockSpec((blk, C_OUT), lambda i: (i, 0)),
          pl.BlockSpec((blk, C_OUT), lambda i: (i, 0)),
          pl.BlockSpec((blk, 1), lambda i: (i, 0)),
          pl.BlockSpec((blk, 1), lambda i: (i, 0)),
      ],
      out_shape=[
          jax.ShapeDtypeStruct((N_PAD, C_OUT), jnp.float32),
          jax.ShapeDtypeStruct((N_PAD, C_OUT), jnp.float32),
          jax.ShapeDtypeStruct((N_PAD, 1), jnp.float32),
          jax.ShapeDtypeStruct((N_PAD, 1), jnp.float32),
      ],
  )(x_pad, W, deg_t)


@functools.partial(
    pl.kernel,
    out_type=jax.ShapeDtypeStruct((2, N_PAD, C_HALF), jnp.float32),
    mesh=_mesh,
    scratch_types=[
        pltpu.VMEM((EROWS_PER_TILE, EC), jnp.int32),
        pltpu.VMEM((EROWS_PER_TILE, EC), jnp.int32),
        pltpu.VMEM((EC, C_HALF), jnp.float32),
        pltpu.VMEM((EC, C_HALF), jnp.float32),
        pltpu.VMEM((EC, C_HALF), jnp.float32),
        pltpu.VMEM((EC, C_HALF), jnp.float32),
        pltpu.VMEM((NPT, C_HALF), jnp.float32),
        pltpu.VMEM((NPT,), jnp.float32),
        pltpu.VMEM_SHARED((N_PAD, C_HALF), jnp.float32),
        pltpu.VMEM_SHARED((N_PAD, C_HALF), jnp.float32),
        pltpu.SemaphoreType.DMA,
    ],
    compiler_params=pltpu.CompilerParams(use_tc_tiling_on_sc=False),
)
def _sc_hops(u0_hbm, row_hbm, col_hbm, d1_hbm, s_out,
             rowb, colb, eb, abuf, ubuf, zbuf, sbuf, d1b, u_sh, acc_sh, sem):
  cid = lax.axis_index("c")
  sid = lax.axis_index("s")
  nbase = sid * NPT
  pltpu.sync_copy(row_hbm.at[pl.ds(sid * EROWS_PER_TILE, EROWS_PER_TILE)], rowb)
  pltpu.sync_copy(col_hbm.at[pl.ds(sid * EROWS_PER_TILE, EROWS_PER_TILE)], colb)
  pltpu.sync_copy(d1_hbm.at[pl.ds(nbase, NPT)], d1b)
  pltpu.sync_copy(u0_hbm.at[cid, pl.ds(nbase, NPT)], u_sh.at[pl.ds(nbase, NPT)])
  _fill2d(zbuf, EC, C_HALF, 0.0, jnp.float32)

  _fill2d(sbuf, NPT, C_HALF, 0.0, jnp.float32)

  def zchunk(j, _):
    pltpu.sync_copy(zbuf, acc_sh.at[pl.ds(nbase + j * EC, EC)])
    return 0
  lax.fori_loop(0, NCH, zchunk, 0)
  plsc.subcore_barrier()

  def k_body(k, _):
    def echunk(j, _):
      pltpu.async_copy(u_sh.at[rowb.at[j]], eb, sem).wait()
      pltpu.sync_copy(eb, acc_sh.at[colb.at[j]], add=True)
      return 0
    lax.fori_loop(0, EROWS_PER_TILE, echunk, 0)
    plsc.subcore_barrier()

    def nchunk(j, _):
      base = nbase + j * EC
      pltpu.sync_copy(acc_sh.at[pl.ds(base, EC)], abuf)
      pltpu.sync_copy(u_sh.at[pl.ds(base, EC)], ubuf)
      pltpu.sync_copy(zbuf, acc_sh.at[pl.ds(base, EC)])

      def rblk(t, _):
        d1vec = d1b[pl.ds(j * EC + t * 16, 16)]
        for rr in range(16):
          r = t * 16 + rr
          dscale = d1vec[rr]
          for g in range(C_HALF // 16):
            sl = pl.ds(g * 16, 16)
            v = (abuf[r, sl] + ubuf[r, sl]) * dscale
            ubuf[r, sl] = v
            sbuf[j * EC + r, sl] = sbuf[j * EC + r, sl] + v
        return 0
      lax.fori_loop(0, EC // 16, rblk, 0)
      pltpu.sync_copy(ubuf, u_sh.at[pl.ds(base, EC)])
      return 0
    lax.fori_loop(0, NCH, nchunk, 0)
    plsc.subcore_barrier()
    return 0
  lax.fori_loop(0, K_HOPS, k_body, 0)
  pltpu.sync_copy(sbuf, s_out.at[cid, pl.ds(nbase, NPT)])


def _tc_final_body(y0_ref, s_ref, dsq_ref, b_ref, o_ref):
  logits = (ALPHA * y0_ref[...] + COEF * dsq_ref[...] * s_ref[...]
            + b_ref[...])
  m = jnp.max(logits, axis=1, keepdims=True)
  ex = jnp.exp(logits - m)
  lse = jnp.log(jnp.sum(ex, axis=1, keepdims=True)) + m
  o_ref[...] = logits - lse


def _tc_final(y0, S, dsq, b2):
  blk = 512
  grid = (N_PAD // blk,)
  return pl.pallas_call(
      _tc_final_body,
      grid=grid,
      in_specs=[
          pl.BlockSpec((blk, C_OUT), lambda i: (i, 0)),
          pl.BlockSpec((blk, C_OUT), lambda i: (i, 0)),
          pl.BlockSpec((blk, 1), lambda i: (i, 0)),
          pl.BlockSpec((1, C_OUT), lambda i: (0, 0)),
      ],
      out_specs=pl.BlockSpec((blk, C_OUT), lambda i: (i, 0)),
      out_shape=jax.ShapeDtypeStruct((N_PAD, C_OUT), jnp.float32),
  )(y0, S, dsq, b2)


def kernel(x, edge_index, W, b):
  row = edge_index[0]
  col = edge_index[1]
  pad = EROWS * EC - E_EDGES
  rowp = jnp.concatenate(
      [row, jnp.full((pad,), N_NODES, jnp.int32)]).reshape(EROWS, EC)
  colp = jnp.concatenate(
      [col, jnp.full((pad,), N_NODES, jnp.int32)]).reshape(EROWS, EC)
  x_pad = jnp.pad(x, ((0, N_PAD - N_NODES), (0, 0)))

  deg_parts = _sc_degree(colp)
  deg_t = jnp.transpose(deg_parts)
  y0, u0, d1, dsq = _tc_prep(x_pad, W, deg_t)
  u0_split = jnp.transpose(u0.reshape(N_PAD, 2, C_HALF), (1, 0, 2))
  d1_flat = d1.reshape(N_PAD)
  s2 = _sc_hops(u0_split, rowp, colp, d1_flat)
  S = jnp.transpose(s2, (1, 0, 2)).reshape(N_PAD, C_OUT)
  out = _tc_final(y0, S, dsq, b.reshape(1, C_OUT))
  return out[:N_NODES]


# speedup vs baseline: 25.4273x; 1.2591x over previous
import functools

import jax
import jax.numpy as jnp
from jax import lax
from jax.experimental import pallas as pl
from jax.experimental.pallas import tpu as pltpu
from jax.experimental.pallas import tpu_sc as plsc

N_NODES = 10000
N_PAD = 10240
D_IN = 128
C_OUT = 64
C_HALF = 32
E_EDGES = 320000
K_HOPS = 16
ALPHA = 0.05
COEF = (1.0 - ALPHA) / K_HOPS

EC = 128
EROWS = 2560
EROWS_PER_TILE = EROWS // 16
EROWS_PER_TILE32 = EROWS // 32
NPT = N_PAD // 16
NCH = NPT // 128

_mesh = plsc.VectorSubcoreMesh(
    core_axis_name="c", subcore_axis_name="s", num_cores=2, num_subcores=16)


def _fill(ref, n16, val, dtype):
  def body(i, _):
    ref[pl.ds(i * 16, 16)] = jnp.full((16,), val, dtype)
    return 0
  lax.fori_loop(0, n16, body, 0)


def _fill2d(ref, rows, cols, val, dtype):
  def body(i, _):
    r = i // (cols // 16)
    g = i % (cols // 16)
    ref[r, pl.ds(g * 16, 16)] = jnp.full((16,), val, dtype)
    return 0
  lax.fori_loop(0, rows * (cols // 16), body, 0)


@functools.partial(
    pl.kernel,
    out_type=jax.ShapeDtypeStruct((2, N_PAD), jnp.float32),
    mesh=_mesh,
    scratch_types=[
        pltpu.VMEM((EROWS_PER_TILE32, EC), jnp.int32),
        pltpu.VMEM((EC,), jnp.float32),
        pltpu.VMEM((NPT,), jnp.float32),
        pltpu.VMEM_SHARED((N_PAD,), jnp.float32),
    ],
)
def _sc_degree(col_hbm, deg_out, colb, ones, zb, degsh):
  cid = lax.axis_index("c")
  sid = lax.axis_index("s")
  tid = sid * 2 + cid
  pltpu.sync_copy(col_hbm.at[pl.ds(tid * EROWS_PER_TILE32, EROWS_PER_TILE32)],
                  colb)
  _fill(ones, EC // 16, 1.0, jnp.float32)
  _fill(zb, NPT // 16, 0.0, jnp.float32)
  pltpu.sync_copy(zb, degsh.at[pl.ds(sid * NPT, NPT)])
  plsc.subcore_barrier()

  def chunk(j, _):
    pltpu.sync_copy(ones, degsh.at[colb.at[j]], add=True)
    return 0
  lax.fori_loop(0, EROWS_PER_TILE32, chunk, 0)
  plsc.subcore_barrier()
  pltpu.sync_copy(degsh.at[pl.ds(sid * NPT, NPT)],
                  deg_out.at[cid, pl.ds(sid * NPT, NPT)])


def _tc_prep_body(x_ref, w_ref, dp_ref, y0_ref, u0_ref, d1_ref, dsq_ref):
  y0 = jnp.dot(x_ref[...], w_ref[...], preferred_element_type=jnp.float32)
  deg = dp_ref[:, 0:1] + dp_ref[:, 1:2] + 1.0
  dinv = lax.rsqrt(deg)
  y0_ref[...] = y0
  u0_ref[...] = y0 * dinv
  d1_ref[...] = dinv * dinv
  dsq_ref[...] = deg * dinv


def _tc_prep(x_pad, W, deg_t):
  blk = 512
  grid = (N_PAD // blk,)
  return pl.pallas_call(
      _tc_prep_body,
      grid=grid,
      in_specs=[
          pl.BlockSpec((blk, D_IN), lambda i: (i, 0)),
          pl.BlockSpec((D_IN, C_OUT), lambda i: (0, 0)),
          pl.BlockSpec((blk, 2), lambda i: (i, 0)),
      ],
      out_specs=[
          pl.BlockSpec((blk, C_OUT), lambda i: (i, 0)),
          pl.BlockSpec((blk, C_OUT), lambda i: (i, 0)),
          pl.BlockSpec((blk, 1), lambda i: (i, 0)),
          pl.BlockSpec((blk, 1), lambda i: (i, 0)),
      ],
      out_shape=[
          jax.ShapeDtypeStruct((N_PAD, C_OUT), jnp.float32),
          jax.ShapeDtypeStruct((N_PAD, C_OUT), jnp.float32),
          jax.ShapeDtypeStruct((N_PAD, 1), jnp.float32),
          jax.ShapeDtypeStruct((N_PAD, 1), jnp.float32),
      ],
  )(x_pad, W, deg_t)


@functools.partial(
    pl.kernel,
    out_type=jax.ShapeDtypeStruct((2, N_PAD, C_HALF), jnp.float32),
    mesh=_mesh,
    scratch_types=[
        pltpu.VMEM((EROWS_PER_TILE + 8, EC), jnp.int32),
        pltpu.VMEM((EROWS_PER_TILE, EC), jnp.int32),
        pltpu.VMEM((EC, C_HALF), jnp.float32),
        pltpu.VMEM((EC, C_HALF), jnp.float32),
        pltpu.VMEM((EC, C_HALF), jnp.float32),
        pltpu.VMEM((EC, C_HALF), jnp.float32),
        pltpu.VMEM((EC, C_HALF), jnp.float32),
        pltpu.VMEM((NPT, C_HALF), jnp.float32),
        pltpu.VMEM((NPT,), jnp.float32),
        pltpu.VMEM_SHARED((N_PAD, C_HALF), jnp.float32),
        pltpu.VMEM_SHARED((N_PAD, C_HALF), jnp.float32),
        pltpu.SemaphoreType.DMA,
        pltpu.SemaphoreType.DMA,
    ],
    compiler_params=pltpu.CompilerParams(use_tc_tiling_on_sc=False),
)
def _sc_hops(u0_hbm, row_hbm, col_hbm, d1_hbm, s_out,
             rowb, colb, eb0, eb1, abuf, ubuf, zbuf, sbuf, d1b, u_sh, acc_sh,
             sem_g, sem_s):
  cid = lax.axis_index("c")
  sid = lax.axis_index("s")
  nbase = sid * NPT
  pltpu.sync_copy(row_hbm.at[pl.ds(sid * EROWS_PER_TILE, EROWS_PER_TILE)],
                  rowb.at[pl.ds(0, EROWS_PER_TILE)])
  def padrow(i, _):
    rowb[EROWS_PER_TILE + i // 8, pl.ds((i % 8) * 16, 16)] = jnp.full(
        (16,), N_NODES, jnp.int32)
    return 0
  lax.fori_loop(0, 8 * 8, padrow, 0)
  pltpu.sync_copy(col_hbm.at[pl.ds(sid * EROWS_PER_TILE, EROWS_PER_TILE)], colb)
  pltpu.sync_copy(d1_hbm.at[pl.ds(nbase, NPT)], d1b)
  pltpu.sync_copy(u0_hbm.at[cid, pl.ds(nbase, NPT)], u_sh.at[pl.ds(nbase, NPT)])
  _fill2d(zbuf, EC, C_HALF, 0.0, jnp.float32)

  _fill2d(sbuf, NPT, C_HALF, 0.0, jnp.float32)

  def zchunk(j, _):
    pltpu.sync_copy(zbuf, acc_sh.at[pl.ds(nbase + j * EC, EC)])
    return 0
  lax.fori_loop(0, NCH, zchunk, 0)
  plsc.subcore_barrier()

  def k_body(k, _):
    def wait_g(buf):
      pltpu.make_async_copy(u_sh.at[rowb.at[0]], buf, sem_g).wait()

    def wait_s(buf):
      pltpu.make_async_copy(buf, acc_sh.at[colb.at[0]], sem_s).wait()

    pltpu.async_copy(u_sh.at[rowb.at[0]], eb0, sem_g)

    def estep(jj, _):
      j0 = 2 * jj
      wait_g(eb0)
      pltpu.async_copy(u_sh.at[rowb.at[j0 + 1]], eb1, sem_g)
      pltpu.async_copy(eb0, acc_sh.at[colb.at[j0]], sem_s, add=True)
      wait_g(eb1)
      wait_s(eb0)
      pltpu.async_copy(u_sh.at[rowb.at[j0 + 2]], eb0, sem_g)
      pltpu.async_copy(eb1, acc_sh.at[colb.at[j0 + 1]], sem_s, add=True)
      wait_s(eb1)
      return 0
    lax.fori_loop(0, EROWS_PER_TILE // 2, estep, 0)
    wait_g(eb0)
    plsc.subcore_barrier()

    def nchunk(j, _):
      base = nbase + j * EC
      pltpu.sync_copy(acc_sh.at[pl.ds(base, EC)], abuf)
      pltpu.sync_copy(u_sh.at[pl.ds(base, EC)], ubuf)
      pltpu.sync_copy(zbuf, acc_sh.at[pl.ds(base, EC)])

      def rblk(t, _):
        d1vec = d1b[pl.ds(j * EC + t * 16, 16)]
        for rr in range(16):
          r = t * 16 + rr
          dscale = d1vec[rr]
          for g in range(C_HALF // 16):
            sl = pl.ds(g * 16, 16)
            v = (abuf[r, sl] + ubuf[r, sl]) * dscale
            ubuf[r, sl] = v
            sbuf[j * EC + r, sl] = sbuf[j * EC + r, sl] + v
        return 0
      lax.fori_loop(0, EC // 16, rblk, 0)
      pltpu.sync_copy(ubuf, u_sh.at[pl.ds(base, EC)])
      return 0
    lax.fori_loop(0, NCH, nchunk, 0)
    plsc.subcore_barrier()
    return 0
  lax.fori_loop(0, K_HOPS, k_body, 0)
  pltpu.sync_copy(sbuf, s_out.at[cid, pl.ds(nbase, NPT)])


def _tc_final_body(y0_ref, s_ref, dsq_ref, b_ref, o_ref):
  logits = (ALPHA * y0_ref[...] + COEF * dsq_ref[...] * s_ref[...]
            + b_ref[...])
  m = jnp.max(logits, axis=1, keepdims=True)
  ex = jnp.exp(logits - m)
  lse = jnp.log(jnp.sum(ex, axis=1, keepdims=True)) + m
  o_ref[...] = logits - lse


def _tc_final(y0, S, dsq, b2):
  blk = 512
  grid = (N_PAD // blk,)
  return pl.pallas_call(
      _tc_final_body,
      grid=grid,
      in_specs=[
          pl.BlockSpec((blk, C_OUT), lambda i: (i, 0)),
          pl.BlockSpec((blk, C_OUT), lambda i: (i, 0)),
          pl.BlockSpec((blk, 1), lambda i: (i, 0)),
          pl.BlockSpec((1, C_OUT), lambda i: (0, 0)),
      ],
      out_specs=pl.BlockSpec((blk, C_OUT), lambda i: (i, 0)),
      out_shape=jax.ShapeDtypeStruct((N_PAD, C_OUT), jnp.float32),
  )(y0, S, dsq, b2)


def kernel(x, edge_index, W, b):
  row = edge_index[0]
  col = edge_index[1]
  pad = EROWS * EC - E_EDGES
  rowp = jnp.concatenate(
      [row, jnp.full((pad,), N_NODES, jnp.int32)]).reshape(EROWS, EC)
  colp = jnp.concatenate(
      [col, jnp.full((pad,), N_NODES, jnp.int32)]).reshape(EROWS, EC)
  x_pad = jnp.pad(x, ((0, N_PAD - N_NODES), (0, 0)))

  deg_parts = _sc_degree(colp)
  deg_t = jnp.transpose(deg_parts)
  y0, u0, d1, dsq = _tc_prep(x_pad, W, deg_t)
  u0_split = jnp.transpose(u0.reshape(N_PAD, 2, C_HALF), (1, 0, 2))
  d1_flat = d1.reshape(N_PAD)
  s2 = _sc_hops(u0_split, rowp, colp, d1_flat)
  S = jnp.transpose(s2, (1, 0, 2)).reshape(N_PAD, C_OUT)
  out = _tc_final(y0, S, dsq, b.reshape(1, C_OUT))
  return out[:N_NODES]


# speedup vs baseline: 27.3418x; 1.0753x over previous
import functools

import jax
import jax.numpy as jnp
from jax import lax
from jax.experimental import pallas as pl
from jax.experimental.pallas import tpu as pltpu
from jax.experimental.pallas import tpu_sc as plsc

N_NODES = 10000
N_PAD = 10240
D_IN = 128
C_OUT = 64
C_HALF = 32
E_EDGES = 320000
K_HOPS = 16
ALPHA = 0.05
COEF = (1.0 - ALPHA) / K_HOPS

EC = 128
EROWS = 2560
EROWS_PER_TILE = EROWS // 16
EROWS_PER_TILE32 = EROWS // 32
NPT = N_PAD // 16
NCH = NPT // 128

_mesh = plsc.VectorSubcoreMesh(
    core_axis_name="c", subcore_axis_name="s", num_cores=2, num_subcores=16)


def _fill(ref, n16, val, dtype):
  def body(i, _):
    ref[pl.ds(i * 16, 16)] = jnp.full((16,), val, dtype)
    return 0
  lax.fori_loop(0, n16, body, 0)


def _fill2d(ref, rows, cols, val, dtype):
  def body(i, _):
    r = i // (cols // 16)
    g = i % (cols // 16)
    ref[r, pl.ds(g * 16, 16)] = jnp.full((16,), val, dtype)
    return 0
  lax.fori_loop(0, rows * (cols // 16), body, 0)


@functools.partial(
    pl.kernel,
    out_type=jax.ShapeDtypeStruct((2, N_PAD), jnp.float32),
    mesh=_mesh,
    scratch_types=[
        pltpu.VMEM((EROWS_PER_TILE32, EC), jnp.int32),
        pltpu.VMEM((EC,), jnp.float32),
        pltpu.VMEM((NPT,), jnp.float32),
        pltpu.VMEM_SHARED((N_PAD,), jnp.float32),
    ],
)
def _sc_degree(col_hbm, deg_out, colb, ones, zb, degsh):
  cid = lax.axis_index("c")
  sid = lax.axis_index("s")
  tid = sid * 2 + cid
  pltpu.sync_copy(col_hbm.at[pl.ds(tid * EROWS_PER_TILE32, EROWS_PER_TILE32)],
                  colb)
  _fill(ones, EC // 16, 1.0, jnp.float32)
  _fill(zb, NPT // 16, 0.0, jnp.float32)
  pltpu.sync_copy(zb, degsh.at[pl.ds(sid * NPT, NPT)])
  plsc.subcore_barrier()

  def chunk(j, _):
    pltpu.sync_copy(ones, degsh.at[colb.at[j]], add=True)
    return 0
  lax.fori_loop(0, EROWS_PER_TILE32, chunk, 0)
  plsc.subcore_barrier()
  pltpu.sync_copy(degsh.at[pl.ds(sid * NPT, NPT)],
                  deg_out.at[cid, pl.ds(sid * NPT, NPT)])


def _tc_prep_body(x_ref, w_ref, dp_ref, y0_ref, u0_ref, d1_ref, dsq_ref):
  y0 = jnp.dot(x_ref[...], w_ref[...], preferred_element_type=jnp.float32)
  deg = dp_ref[:, 0:1] + dp_ref[:, 1:2] + 1.0
  dinv = lax.rsqrt(deg)
  y0_ref[...] = y0
  u0_ref[...] = y0 * dinv
  d1_ref[...] = dinv * dinv
  dsq_ref[...] = deg * dinv


def _tc_prep(x_pad, W, deg_t):
  blk = 512
  grid = (N_PAD // blk,)
  return pl.pallas_call(
      _tc_prep_body,
      grid=grid,
      in_specs=[
          pl.BlockSpec((blk, D_IN), lambda i: (i, 0)),
          pl.BlockSpec((D_IN, C_OUT), lambda i: (0, 0)),
          pl.BlockSpec((blk, 2), lambda i: (i, 0)),
      ],
      out_specs=[
          pl.BlockSpec((blk, C_OUT), lambda i: (i, 0)),
          pl.BlockSpec((blk, C_OUT), lambda i: (i, 0)),
          pl.BlockSpec((blk, 1), lambda i: (i, 0)),
          pl.BlockSpec((blk, 1), lambda i: (i, 0)),
      ],
      out_shape=[
          jax.ShapeDtypeStruct((N_PAD, C_OUT), jnp.float32),
          jax.ShapeDtypeStruct((N_PAD, C_OUT), jnp.float32),
          jax.ShapeDtypeStruct((N_PAD, 1), jnp.float32),
          jax.ShapeDtypeStruct((N_PAD, 1), jnp.float32),
      ],
  )(x_pad, W, deg_t)


@functools.partial(
    pl.kernel,
    out_type=jax.ShapeDtypeStruct((2, N_PAD, C_HALF), jnp.float32),
    mesh=_mesh,
    scratch_types=[
        pltpu.VMEM((EROWS_PER_TILE + 8, EC), jnp.int32),
        pltpu.VMEM((EROWS_PER_TILE, EC), jnp.int32),
        pltpu.VMEM((EC, C_HALF), jnp.float32),
        pltpu.VMEM((EC, C_HALF), jnp.float32),
        pltpu.VMEM((EC, C_HALF), jnp.float32),
        pltpu.VMEM((EC, C_HALF), jnp.float32),
        pltpu.VMEM((EC, C_HALF), jnp.float32),
        pltpu.VMEM((NPT, C_HALF), jnp.float32),
        pltpu.VMEM((NPT,), jnp.float32),
        pltpu.VMEM_SHARED((N_PAD, C_HALF), jnp.float32),
        pltpu.VMEM_SHARED((N_PAD, C_HALF), jnp.float32),
        pltpu.SemaphoreType.DMA,
        pltpu.SemaphoreType.DMA,
    ],
    compiler_params=pltpu.CompilerParams(use_tc_tiling_on_sc=False),
)
def _sc_hops(u0_hbm, row_hbm, col_hbm, d1_hbm, s_out,
             rowb, colb, eb0, eb1, eb2, eb3, zbuf, sbuf, d1b,
             u_sh, acc_sh, sem_g, sem_s):
  abuf, ubuf = eb2, eb3
  cid = lax.axis_index("c")
  sid = lax.axis_index("s")
  nbase = sid * NPT
  pltpu.sync_copy(row_hbm.at[pl.ds(sid * EROWS_PER_TILE, EROWS_PER_TILE)],
                  rowb.at[pl.ds(0, EROWS_PER_TILE)])
  def padrow(i, _):
    rowb[EROWS_PER_TILE + i // 8, pl.ds((i % 8) * 16, 16)] = jnp.full(
        (16,), N_NODES, jnp.int32)
    return 0
  lax.fori_loop(0, 8 * 8, padrow, 0)
  pltpu.sync_copy(col_hbm.at[pl.ds(sid * EROWS_PER_TILE, EROWS_PER_TILE)], colb)
  pltpu.sync_copy(d1_hbm.at[pl.ds(nbase, NPT)], d1b)
  pltpu.sync_copy(u0_hbm.at[cid, pl.ds(nbase, NPT)], u_sh.at[pl.ds(nbase, NPT)])
  _fill2d(zbuf, EC, C_HALF, 0.0, jnp.float32)

  _fill2d(sbuf, NPT, C_HALF, 0.0, jnp.float32)

  def zchunk(j, _):
    pltpu.sync_copy(zbuf, acc_sh.at[pl.ds(nbase + j * EC, EC)])
    return 0
  lax.fori_loop(0, NCH, zchunk, 0)
  plsc.subcore_barrier()

  def k_body(k, _):
    def wait_g(buf):
      pltpu.make_async_copy(u_sh.at[rowb.at[0]], buf, sem_g).wait()

    def wait_s(buf):
      pltpu.make_async_copy(buf, acc_sh.at[colb.at[0]], sem_s).wait()

    def gath(j, buf):
      pltpu.async_copy(u_sh.at[rowb.at[j]], buf, sem_g)

    def scat(j, buf):
      pltpu.async_copy(buf, acc_sh.at[colb.at[j]], sem_s, add=True)

    gath(0, eb0)
    gath(1, eb1)

    def estep(jj, _):
      j0 = 4 * jj
      wait_g(eb0)
      gath(j0 + 2, eb2)
      scat(j0, eb0)
      wait_g(eb1)
      gath(j0 + 3, eb3)
      scat(j0 + 1, eb1)
      wait_s(eb0)
      gath(j0 + 4, eb0)
      wait_g(eb2)
      scat(j0 + 2, eb2)
      wait_s(eb1)
      gath(j0 + 5, eb1)
      wait_g(eb3)
      scat(j0 + 3, eb3)
      wait_s(eb2)
      wait_s(eb3)
      return 0
    lax.fori_loop(0, EROWS_PER_TILE // 4, estep, 0)
    wait_g(eb0)
    wait_g(eb1)
    plsc.subcore_barrier()

    def nchunk(j, _):
      base = nbase + j * EC
      pltpu.sync_copy(acc_sh.at[pl.ds(base, EC)], abuf)
      pltpu.sync_copy(u_sh.at[pl.ds(base, EC)], ubuf)
      pltpu.sync_copy(zbuf, acc_sh.at[pl.ds(base, EC)])

      def rblk(t, _):
        d1vec = d1b[pl.ds(j * EC + t * 16, 16)]
        for rr in range(16):
          r = t * 16 + rr
          dscale = d1vec[rr]
          for g in range(C_HALF // 16):
            sl = pl.ds(g * 16, 16)
            v = (abuf[r, sl] + ubuf[r, sl]) * dscale
            ubuf[r, sl] = v
            sbuf[j * EC + r, sl] = sbuf[j * EC + r, sl] + v
        return 0
      lax.fori_loop(0, EC // 16, rblk, 0)
      pltpu.sync_copy(ubuf, u_sh.at[pl.ds(base, EC)])
      return 0
    lax.fori_loop(0, NCH, nchunk, 0)
    plsc.subcore_barrier()
    return 0
  lax.fori_loop(0, K_HOPS, k_body, 0)
  pltpu.sync_copy(sbuf, s_out.at[cid, pl.ds(nbase, NPT)])


def _tc_final_body(y0_ref, s_ref, dsq_ref, b_ref, o_ref):
  logits = (ALPHA * y0_ref[...] + COEF * dsq_ref[...] * s_ref[...]
            + b_ref[...])
  m = jnp.max(logits, axis=1, keepdims=True)
  ex = jnp.exp(logits - m)
  lse = jnp.log(jnp.sum(ex, axis=1, keepdims=True)) + m
  o_ref[...] = logits - lse


def _tc_final(y0, S, dsq, b2):
  blk = 512
  grid = (N_PAD // blk,)
  return pl.pallas_call(
      _tc_final_body,
      grid=grid,
      in_specs=[
          pl.BlockSpec((blk, C_OUT), lambda i: (i, 0)),
          pl.BlockSpec((blk, C_OUT), lambda i: (i, 0)),
          pl.BlockSpec((blk, 1), lambda i: (i, 0)),
          pl.BlockSpec((1, C_OUT), lambda i: (0, 0)),
      ],
      out_specs=pl.BlockSpec((blk, C_OUT), lambda i: (i, 0)),
      out_shape=jax.ShapeDtypeStruct((N_PAD, C_OUT), jnp.float32),
  )(y0, S, dsq, b2)


def kernel(x, edge_index, W, b):
  row = edge_index[0]
  col = edge_index[1]
  pad = EROWS * EC - E_EDGES
  rowp = jnp.concatenate(
      [row, jnp.full((pad,), N_NODES, jnp.int32)]).reshape(EROWS, EC)
  colp = jnp.concatenate(
      [col, jnp.full((pad,), N_NODES, jnp.int32)]).reshape(EROWS, EC)
  x_pad = jnp.pad(x, ((0, N_PAD - N_NODES), (0, 0)))

  deg_parts = _sc_degree(colp)
  deg_t = jnp.transpose(deg_parts)
  y0, u0, d1, dsq = _tc_prep(x_pad, W, deg_t)
  u0_split = jnp.transpose(u0.reshape(N_PAD, 2, C_HALF), (1, 0, 2))
  d1_flat = d1.reshape(N_PAD)
  s2 = _sc_hops(u0_split, rowp, colp, d1_flat)
  S = jnp.transpose(s2, (1, 0, 2)).reshape(N_PAD, C_OUT)
  out = _tc_final(y0, S, dsq, b.reshape(1, C_OUT))
  return out[:N_NODES]


# speedup vs baseline: 44.0182x; 1.6099x over previous
import functools

import jax
import jax.numpy as jnp
from jax import lax
from jax.experimental import pallas as pl
from jax.experimental.pallas import tpu as pltpu
from jax.experimental.pallas import tpu_sc as plsc

N_NODES = 10000
N_PAD = 10240
D_IN = 128
C_OUT = 64
C_HALF = 32
E_EDGES = 320000
K_HOPS = 16
ALPHA = 0.05
COEF = (1.0 - ALPHA) / K_HOPS

EC = 128
EROWS = 2560
EROWS_PER_TILE = EROWS // 16
EROWS_PER_TILE32 = EROWS // 32
NPT = N_PAD // 16
NCH = NPT // 128

_mesh = plsc.VectorSubcoreMesh(
    core_axis_name="c", subcore_axis_name="s", num_cores=2, num_subcores=16)


def _fill(ref, n16, val, dtype):
  def body(i, _):
    ref[pl.ds(i * 16, 16)] = jnp.full((16,), val, dtype)
    return 0
  lax.fori_loop(0, n16, body, 0)


def _fill2d(ref, rows, cols, val, dtype):
  def body(i, _):
    r = i // (cols // 16)
    g = i % (cols // 16)
    ref[r, pl.ds(g * 16, 16)] = jnp.full((16,), val, dtype)
    return 0
  lax.fori_loop(0, rows * (cols // 16), body, 0)


@functools.partial(
    pl.kernel,
    out_type=jax.ShapeDtypeStruct((2, N_PAD), jnp.float32),
    mesh=_mesh,
    scratch_types=[
        pltpu.VMEM((EROWS_PER_TILE32, EC), jnp.int32),
        pltpu.VMEM((EC,), jnp.float32),
        pltpu.VMEM((NPT,), jnp.float32),
        pltpu.VMEM_SHARED((N_PAD,), jnp.float32),
    ],
)
def _sc_degree(col_hbm, deg_out, colb, ones, zb, degsh):
  cid = lax.axis_index("c")
  sid = lax.axis_index("s")
  tid = sid * 2 + cid
  pltpu.sync_copy(col_hbm.at[pl.ds(tid * EROWS_PER_TILE32, EROWS_PER_TILE32)],
                  colb)
  _fill(ones, EC // 16, 1.0, jnp.float32)
  _fill(zb, NPT // 16, 0.0, jnp.float32)
  pltpu.sync_copy(zb, degsh.at[pl.ds(sid * NPT, NPT)])
  plsc.subcore_barrier()

  def chunk(j, _):
    pltpu.sync_copy(ones, degsh.at[colb.at[j]], add=True)
    return 0
  lax.fori_loop(0, EROWS_PER_TILE32, chunk, 0)
  plsc.subcore_barrier()
  pltpu.sync_copy(degsh.at[pl.ds(sid * NPT, NPT)],
                  deg_out.at[cid, pl.ds(sid * NPT, NPT)])


def _tc_prep_body(x_ref, w_ref, dp_ref, y0_ref, u0_ref, d1_ref, dsq_ref):
  y0 = jnp.dot(x_ref[...], w_ref[...], preferred_element_type=jnp.float32)
  deg = dp_ref[:, 0:1] + dp_ref[:, 1:2] + 1.0
  dinv = lax.rsqrt(deg)
  y0_ref[...] = y0
  u0_ref[...] = (y0 * dinv).astype(jnp.bfloat16)
  d1_ref[...] = dinv * dinv
  dsq_ref[...] = deg * dinv


def _tc_prep(x_pad, W, deg_t):
  blk = 512
  grid = (N_PAD // blk,)
  return pl.pallas_call(
      _tc_prep_body,
      grid=grid,
      in_specs=[
          pl.BlockSpec((blk, D_IN), lambda i: (i, 0)),
          pl.BlockSpec((D_IN, C_OUT), lambda i: (0, 0)),
          pl.BlockSpec((blk, 2), lambda i: (i, 0)),
      ],
      out_specs=[
          pl.BlockSpec((blk, C_OUT), lambda i: (i, 0)),
          pl.BlockSpec((blk, C_OUT), lambda i: (i, 0)),
          pl.BlockSpec((blk, 1), lambda i: (i, 0)),
          pl.BlockSpec((blk, 1), lambda i: (i, 0)),
      ],
      out_shape=[
          jax.ShapeDtypeStruct((N_PAD, C_OUT), jnp.float32),
          jax.ShapeDtypeStruct((N_PAD, C_OUT), jnp.bfloat16),
          jax.ShapeDtypeStruct((N_PAD, 1), jnp.float32),
          jax.ShapeDtypeStruct((N_PAD, 1), jnp.float32),
      ],
  )(x_pad, W, deg_t)


@functools.partial(
    pl.kernel,
    out_type=jax.ShapeDtypeStruct((2, N_PAD, C_HALF), jnp.float32),
    mesh=_mesh,
    scratch_types=[
        pltpu.VMEM((EROWS_PER_TILE + 8, EC), jnp.int32),
        pltpu.VMEM((EROWS_PER_TILE, EC), jnp.int32),
        pltpu.VMEM((EC, C_HALF), jnp.bfloat16),
        pltpu.VMEM((EC, C_HALF), jnp.bfloat16),
        pltpu.VMEM((EC, C_HALF), jnp.bfloat16),
        pltpu.VMEM((EC, C_HALF), jnp.bfloat16),
        pltpu.VMEM((EC, C_HALF), jnp.bfloat16),
        pltpu.VMEM((NPT, C_HALF), jnp.float32),
        pltpu.VMEM((NPT,), jnp.float32),
        pltpu.VMEM_SHARED((N_PAD, C_HALF), jnp.bfloat16),
        pltpu.VMEM_SHARED((N_PAD, C_HALF), jnp.bfloat16),
        pltpu.SemaphoreType.DMA,
        pltpu.SemaphoreType.DMA,
    ],
    compiler_params=pltpu.CompilerParams(
        use_tc_tiling_on_sc=False, needs_layout_passes=False),
)
def _sc_hops(u0_hbm, row_hbm, col_hbm, d1_hbm, s_out,
             rowb, colb, eb0, eb1, eb2, eb3, zbuf, sbuf, d1b,
             u_sh, acc_sh, sem_g, sem_s):
  abuf, ubuf = eb2, eb3
  cid = lax.axis_index("c")
  sid = lax.axis_index("s")
  nbase = sid * NPT
  pltpu.sync_copy(row_hbm.at[pl.ds(sid * EROWS_PER_TILE, EROWS_PER_TILE)],
                  rowb.at[pl.ds(0, EROWS_PER_TILE)])
  def padrow(i, _):
    rowb[EROWS_PER_TILE + i // 8, pl.ds((i % 8) * 16, 16)] = jnp.full(
        (16,), N_NODES, jnp.int32)
    return 0
  lax.fori_loop(0, 8 * 8, padrow, 0)
  pltpu.sync_copy(col_hbm.at[pl.ds(sid * EROWS_PER_TILE, EROWS_PER_TILE)], colb)
  pltpu.sync_copy(d1_hbm.at[pl.ds(nbase, NPT)], d1b)
  pltpu.sync_copy(u0_hbm.at[cid, pl.ds(nbase, NPT)], u_sh.at[pl.ds(nbase, NPT)])
  def zrow(i, _):
    zbuf[i, :] = jnp.zeros((C_HALF,), jnp.bfloat16)
    return 0
  lax.fori_loop(0, EC, zrow, 0)
  _fill2d(sbuf, NPT, C_HALF, 0.0, jnp.float32)

  def zchunk(j, _):
    pltpu.sync_copy(zbuf, acc_sh.at[pl.ds(nbase + j * EC, EC)])
    return 0
  lax.fori_loop(0, NCH, zchunk, 0)
  plsc.subcore_barrier()

  def k_body(k, _):
    def wait_g(buf):
      pltpu.make_async_copy(u_sh.at[rowb.at[0]], buf, sem_g).wait()

    def wait_s(buf):
      pltpu.make_async_copy(buf, acc_sh.at[colb.at[0]], sem_s).wait()

    def gath(j, buf):
      pltpu.async_copy(u_sh.at[rowb.at[j]], buf, sem_g)

    def scat(j, buf):
      pltpu.async_copy(buf, acc_sh.at[colb.at[j]], sem_s, add=True)

    gath(0, eb0)
    gath(1, eb1)

    def estep(jj, _):
      j0 = 4 * jj
      wait_g(eb0)
      gath(j0 + 2, eb2)
      scat(j0, eb0)
      wait_g(eb1)
      gath(j0 + 3, eb3)
      scat(j0 + 1, eb1)
      wait_s(eb0)
      gath(j0 + 4, eb0)
      wait_g(eb2)
      scat(j0 + 2, eb2)
      wait_s(eb1)
      gath(j0 + 5, eb1)
      wait_g(eb3)
      scat(j0 + 3, eb3)
      wait_s(eb2)
      wait_s(eb3)
      return 0
    lax.fori_loop(0, EROWS_PER_TILE // 4, estep, 0)
    wait_g(eb0)
    wait_g(eb1)
    plsc.subcore_barrier()

    def nchunk(j, _):
      base = nbase + j * EC
      pltpu.sync_copy(acc_sh.at[pl.ds(base, EC)], abuf)
      pltpu.sync_copy(u_sh.at[pl.ds(base, EC)], ubuf)
      pltpu.sync_copy(zbuf, acc_sh.at[pl.ds(base, EC)])

      def rblk(t, _):
        d1vec = d1b[pl.ds(j * EC + t * 16, 16)]
        for rr in range(16):
          r = t * 16 + rr
          dscale = d1vec[rr]
          a0, a1 = plsc.unpack(abuf[r, :], format=plsc.PackFormat.INTERLEAVED)
          b0, b1 = plsc.unpack(ubuf[r, :], format=plsc.PackFormat.INTERLEAVED)
          v0 = (a0 + b0) * dscale
          v1 = (a1 + b1) * dscale
          ubuf[r, :] = plsc.pack(v0, v1, format=plsc.PackFormat.INTERLEAVED)
          sl0 = pl.ds(0, 16)
          sl1 = pl.ds(16, 16)
          sbuf[j * EC + r, sl0] = sbuf[j * EC + r, sl0] + v0
          sbuf[j * EC + r, sl1] = sbuf[j * EC + r, sl1] + v1
        return 0
      lax.fori_loop(0, EC // 16, rblk, 0)
      pltpu.sync_copy(ubuf, u_sh.at[pl.ds(base, EC)])
      return 0
    lax.fori_loop(0, NCH, nchunk, 0)
    plsc.subcore_barrier()
    return 0
  lax.fori_loop(0, K_HOPS, k_body, 0)
  pltpu.sync_copy(sbuf, s_out.at[cid, pl.ds(nbase, NPT)])


def _tc_final_body(y0_ref, s_ref, dsq_ref, b_ref, o_ref):
  logits = (ALPHA * y0_ref[...] + COEF * dsq_ref[...] * s_ref[...]
            + b_ref[...])
  m = jnp.max(logits, axis=1, keepdims=True)
  ex = jnp.exp(logits - m)
  lse = jnp.log(jnp.sum(ex, axis=1, keepdims=True)) + m
  o_ref[...] = logits - lse


def _tc_final(y0, S, dsq, b2):
  blk = 512
  grid = (N_PAD // blk,)
  return pl.pallas_call(
      _tc_final_body,
      grid=grid,
      in_specs=[
          pl.BlockSpec((blk, C_OUT), lambda i: (i, 0)),
          pl.BlockSpec((blk, C_OUT), lambda i: (i, 0)),
          pl.BlockSpec((blk, 1), lambda i: (i, 0)),
          pl.BlockSpec((1, C_OUT), lambda i: (0, 0)),
      ],
      out_specs=pl.BlockSpec((blk, C_OUT), lambda i: (i, 0)),
      out_shape=jax.ShapeDtypeStruct((N_PAD, C_OUT), jnp.float32),
  )(y0, S, dsq, b2)


def kernel(x, edge_index, W, b):
  row = edge_index[0]
  col = edge_index[1]
  pad = EROWS * EC - E_EDGES
  rowp = jnp.concatenate(
      [row, jnp.full((pad,), N_NODES, jnp.int32)]).reshape(EROWS, EC)
  colp = jnp.concatenate(
      [col, jnp.full((pad,), N_NODES, jnp.int32)]).reshape(EROWS, EC)
  x_pad = jnp.pad(x, ((0, N_PAD - N_NODES), (0, 0)))

  deg_parts = _sc_degree(colp)
  deg_t = jnp.transpose(deg_parts)
  y0, u0, d1, dsq = _tc_prep(x_pad, W, deg_t)
  u0_split = jnp.transpose(u0.reshape(N_PAD, 2, C_HALF), (1, 0, 2))
  d1_flat = d1.reshape(N_PAD)
  s2 = _sc_hops(u0_split, rowp, colp, d1_flat)
  s2 = s2.reshape(2, N_PAD, 2, 16).transpose(0, 1, 3, 2).reshape(
      2, N_PAD, C_HALF)
  S = jnp.transpose(s2, (1, 0, 2)).reshape(N_PAD, C_OUT)
  out = _tc_final(y0, S, dsq, b.reshape(1, C_OUT))
  return out[:N_NODES]
